# baseline (speedup 1.0000x reference)
import functools

import jax
import jax.numpy as jnp
from jax import lax
from jax.experimental import pallas as pl
from jax.experimental.pallas import tpu as pltpu
from jax.experimental.pallas import tpu_sc as plsc

N = 10000
NPAD = 10240
NCHILD = 320000
H = 128
D = 128
NC = 2
NS = 16
NW = NC * NS
BLK = 128
NBLK = NCHILD // BLK
ROWS_W = NPAD // NW
ACH = 80
ROWS_T = NPAD // NS
BLKS_PW = 80
CPW = BLKS_PW * BLK
NBLK_PAD = NW * BLKS_PW
NCHILD_PAD = NBLK_PAD * BLK

@functools.lru_cache(maxsize=None)
def _mesh():
    return plsc.VectorSubcoreMesh(core_axis_name="c", subcore_axis_name="s",
                                  num_cores=NC, num_subcores=NS)


def _zero_buf(buf):
    z = jnp.zeros((16,), jnp.float32)

    def row(r, _):
        for v in range(H // 16):
            buf[r, pl.ds(v * 16, 16)] = z
        return 0

    lax.fori_loop(0, BLK, row, 0)


def _zero_acc(acc, buf, s):
    _zero_buf(buf)
    for k in range(ROWS_T // BLK):
        pltpu.sync_copy(buf, acc.at[pl.ds(s * ROWS_T + k * BLK, BLK), :])


@functools.lru_cache(maxsize=None)
def _gather_x_kernel():
    return pl.kernel(
        _gather_x_body,
        out_type=jax.ShapeDtypeStruct((NPAD, D), jnp.float32),
        mesh=_mesh(),
        scratch_types=[
            pltpu.VMEM((ROWS_W,), jnp.int32),
            pltpu.VMEM((ROWS_W,), jnp.int32),
            pltpu.VMEM((ACH, D), jnp.float32),
            pltpu.SemaphoreType.DMA,
        ],
    )


def _gather_x_body(emb, sen, sel, x_out, sel_v, senv_v, rows_v, sem):
    c = lax.axis_index("c")
    s = lax.axis_index("s")
    w = s * NC + c
    base = w * ROWS_W
    pltpu.sync_copy(sel.at[pl.ds(base, ROWS_W)], sel_v)
    for k in range(ROWS_W // ACH):
        pltpu.async_copy(
            sen.at[sel_v.at[pl.ds(k * ACH, ACH)]],
            senv_v.at[pl.ds(k * ACH, ACH)],
            sem,
        ).wait()
    for k in range(ROWS_W // ACH):
        pltpu.async_copy(
            emb.at[senv_v.at[pl.ds(k * ACH, ACH)]], rows_v, sem
        ).wait()
        pltpu.sync_copy(rows_v, x_out.at[pl.ds(base + k * ACH, ACH), :])


@functools.lru_cache(maxsize=None)
def _expand_kernel():
    return pl.kernel(
        _expand_body,
        out_type=jax.ShapeDtypeStruct((NCHILD, H), jnp.float32),
        mesh=_mesh(),
        scratch_types=[
            pltpu.VMEM_SHARED((NPAD, H), jnp.float32),
            pltpu.VMEM((CPW,), jnp.int32),
            pltpu.VMEM((BLK, H), jnp.float32),
            pltpu.VMEM((BLK, H), jnp.float32),
            pltpu.SemaphoreType.DMA,
            pltpu.SemaphoreType.DMA,
            pltpu.SemaphoreType.DMA,
            pltpu.SemaphoreType.DMA,
        ],
    )


def _expand_body(fx_all, pid_pad, fxg_out, fx_sh, idx_v, fx0, fx1,
                 sg0, sg1, sw0, sw1):
    c = lax.axis_index("c")
    s = lax.axis_index("s")
    w = s * NC + c
    base = w * BLKS_PW
    nvalid = jnp.clip(NBLK - base, 0, BLKS_PW)
    pltpu.sync_copy(
        fx_all.at[pl.ds(s * ROWS_T, ROWS_T), :],
        fx_sh.at[pl.ds(s * ROWS_T, ROWS_T), :],
    )
    pltpu.sync_copy(pid_pad.at[pl.ds(w * CPW, CPW)], idx_v)
    plsc.subcore_barrier()
    fxb = (fx0, fx1)
    sg = (sg0, sg1)
    sw = (sw0, sw1)
    NBUF = 2
    LEAD = 1

    def gather_desc(t, b):
        return pltpu.make_async_copy(
            fx_sh.at[idx_v.at[pl.ds(t * BLK, BLK)]], fxb[b], sg[b])

    def write_desc(t, b):
        return pltpu.make_async_copy(
            fxb[b], fxg_out.at[pl.ds((base + t) * BLK, BLK), :], sw[b])

    def step(t, b):
        @pl.when(jnp.logical_and(t >= NBUF, t - NBUF < nvalid))
        def _():
            write_desc(t - NBUF, b).wait()

        @pl.when(t < nvalid)
        def _():
            gather_desc(t, b).start()

        bb = (b - LEAD) % NBUF

        @pl.when(jnp.logical_and(t >= LEAD, t - LEAD < nvalid))
        def _():
            gather_desc(t - LEAD, bb).wait()
            write_desc(t - LEAD, bb).start()

    nwave = BLKS_PW + NBUF
    nit = (nwave + NBUF - 1) // NBUF

    def body(j2, _):
        for b in range(NBUF):
            step(j2 * NBUF + b, b)
        return 0

    lax.fori_loop(0, nit, body, 0)


@functools.lru_cache(maxsize=None)
def _segsum_kernel():
    return pl.kernel(
        _segsum_body,
        out_type=jax.ShapeDtypeStruct((NC, NPAD, H), jnp.float32),
        mesh=_mesh(),
        scratch_types=[
            pltpu.VMEM_SHARED((NPAD, H), jnp.float32),
            pltpu.VMEM((BLKS_PW, 1, BLK), jnp.int32),
            pltpu.VMEM((BLK, H), jnp.float32),
            pltpu.VMEM((BLK, H), jnp.float32),
            pltpu.SemaphoreType.DMA,
            pltpu.SemaphoreType.DMA,
            pltpu.SemaphoreType.DMA,
            pltpu.SemaphoreType.DMA,
        ],
    )


def _segsum_body(data, pid3d, out, acc, idx2d, d0, d1, si0, si1, sa0, sa1):
    c = lax.axis_index("c")
    s = lax.axis_index("s")
    w = s * NC + c
    base = w * BLKS_PW
    nvalid = jnp.clip(NBLK - base, 0, BLKS_PW)
    _zero_acc(acc, d0, s)
    pltpu.sync_copy(pid3d.at[pl.ds(base, BLKS_PW), :, :], idx2d)
    plsc.subcore_barrier()
    bufs = (d0, d1)
    si = (si0, si1)
    sa = (sa0, sa1)
    NBUF = 2

    def load_desc(t, b):
        return pltpu.make_async_copy(
            data.at[pl.ds((base + t) * BLK, BLK), :], bufs[b], si[b])

    def add_desc(t, b):
        return pltpu.make_async_copy(bufs[b], acc.at[idx2d.at[t, 0]], sa[b])

    def step(t, b):
        @pl.when(jnp.logical_and(t >= NBUF, t - NBUF < nvalid))
        def _():
            add_desc(t - NBUF, b).wait()

        @pl.when(t < nvalid)
        def _():
            load_desc(t, b).start()

        bb = (b - 1) % NBUF

        @pl.when(jnp.logical_and(t >= 1, t - 1 < nvalid))
        def _():
            load_desc(t - 1, bb).wait()
            pltpu.async_copy(bufs[bb], acc.at[idx2d.at[t - 1, 0]], sa[bb],
                             add=True)

    nwave = BLKS_PW + NBUF
    nit = (nwave + NBUF - 1) // NBUF

    def body(j2, _):
        for b in range(NBUF):
            step(j2 * NBUF + b, b)
        return 0

    lax.fori_loop(0, nit, body, 0)
    plsc.subcore_barrier()
    pltpu.sync_copy(
        acc.at[pl.ds(s * ROWS_T, ROWS_T), :],
        out.at[c, pl.ds(s * ROWS_T, ROWS_T), :],
    )


def _mm_t(a, w):
    return lax.dot_general(a, w, (((1,), (1,)), ((), ())),
                           preferred_element_type=jnp.float32)


def _proj_body(x_ref, w_ref, b_ref, o_ref):
    o_ref[...] = _mm_t(x_ref[...], w_ref[...]) + b_ref[...]


def _fc_body(ch_ref, fxg_ref, cc_ref, w_ref, b_ref, fc_ref):
    z = _mm_t(ch_ref[...], w_ref[...]) + b_ref[...] + fxg_ref[...]
    fc_ref[...] = cc_ref[...] * jax.nn.sigmoid(z)


def _gates_body(x_ref, hsp_ref, fcsp_ref, wix, bix, wih, bih, wox, box, woh,
                boh, wux, bux, wuh, buh, wo, bo, out_ref, c_ref, h_ref):
    xs = x_ref[...]
    hs = hsp_ref[0] + hsp_ref[1]
    fcs = fcsp_ref[0] + fcsp_ref[1]
    i = jax.nn.sigmoid(_mm_t(xs, wix[...]) + bix[...]
                       + _mm_t(hs, wih[...]) + bih[...])
    o = jax.nn.sigmoid(_mm_t(xs, wox[...]) + box[...]
                       + _mm_t(hs, woh[...]) + boh[...])
    u = jnp.tanh(_mm_t(xs, wux[...]) + bux[...]
                 + _mm_t(hs, wuh[...]) + buh[...])
    cc = i * u + fcs
    hh = o * jnp.tanh(cc)
    out_ref[...] = _mm_t(hh, wo[...])
    c_ref[...] = cc
    h_ref[...] = hh


_W_SPEC = pl.BlockSpec((H, H), lambda i: (0, 0))
_B_SPEC = pl.BlockSpec((H,), lambda i: (0,))


def _proj(x, w, b):
    blk = 1280
    return pl.pallas_call(
        _proj_body,
        grid=(NPAD // blk,),
        in_specs=[pl.BlockSpec((blk, D), lambda i: (i, 0)), _W_SPEC, _B_SPEC],
        out_specs=pl.BlockSpec((blk, H), lambda i: (i, 0)),
        out_shape=jax.ShapeDtypeStruct((NPAD, H), jnp.float32),
    )(x, w, b)


def _fc(ch, fxg, cc, w, b):
    blk = 1280
    spec = pl.BlockSpec((blk, H), lambda i: (i, 0))
    return pl.pallas_call(
        _fc_body,
        grid=(NCHILD // blk,),
        in_specs=[spec, spec, spec, _W_SPEC, _B_SPEC],
        out_specs=spec,
        out_shape=jax.ShapeDtypeStruct((NCHILD, H), jnp.float32),
    )(ch, fxg, cc, w, b)


def _gates(x, hsp, fcsp, wix, bix, wih, bih, wox, box, woh, boh, wux, bux,
           wuh, buh, wo, bo):
    blk = 1280
    row_spec = pl.BlockSpec((blk, H), lambda i: (i, 0))
    part_spec = pl.BlockSpec((NC, blk, H), lambda i: (0, i, 0))
    w_specs = [_W_SPEC, _B_SPEC] * 6 + [_W_SPEC, _B_SPEC]
    return pl.pallas_call(
        _gates_body,
        grid=(NPAD // blk,),
        in_specs=[row_spec, part_spec, part_spec] + w_specs,
        out_specs=(row_spec, row_spec, row_spec),
        out_shape=(
            jax.ShapeDtypeStruct((NPAD, H), jnp.float32),
            jax.ShapeDtypeStruct((NPAD, H), jnp.float32),
            jax.ShapeDtypeStruct((NPAD, H), jnp.float32),
        ),
    )(x, hsp, fcsp, wix, bix, wih, bih, wox, box, woh, boh, wux, bux, wuh,
      buh, wo, bo)


def kernel(embedding, Wix, bix, Wih, bih, Wfx, bfx, Wfh, bfh,
           Wox, box, Woh, boh, Wux, bux, Wuh, buh, Wout, bout,
           child_h, child_c, sen, select_indices, parent_ids):
    sen32 = sen.astype(jnp.int32)
    sel32 = jnp.pad(select_indices.astype(jnp.int32), (0, NPAD - N))
    pid32 = parent_ids.astype(jnp.int32)
    wo_pad = jnp.pad(Wout, ((0, H - Wout.shape[0]), (0, 0)))

    pid_pad = jnp.pad(pid32, (0, NCHILD_PAD - NCHILD))
    pid3d = pid_pad.reshape(NBLK_PAD, 1, BLK)

    hs_parts = _segsum_kernel()(child_h, pid3d)
    x = _gather_x_kernel()(embedding, sen32, sel32)
    fx_all = _proj(x, Wfx, bfx)
    fxg = _expand_kernel()(fx_all, pid_pad)
    fc = _fc(child_h, fxg, child_c, Wfh, bfh)
    fcs_parts = _segsum_kernel()(fc, pid3d)
    out_full, c, h = _gates(x, hs_parts, fcs_parts, Wix, bix, Wih, bih,
                            Wox, box, Woh, boh, Wux, bux, Wuh, buh,
                            wo_pad, bout)
    out = out_full[:N, : bout.shape[0]] + bout
    return out, c[:N], h[:N]


# speedup vs baseline: 1.0084x; 1.0084x over previous
import functools

import jax
import jax.numpy as jnp
from jax import lax
from jax.experimental import pallas as pl
from jax.experimental.pallas import tpu as pltpu
from jax.experimental.pallas import tpu_sc as plsc

N = 10000
NPAD = 10240
NCHILD = 320000
H = 128
D = 128
NC = 2
NS = 16
NW = NC * NS
BLK = 128
NBLK = NCHILD // BLK
ROWS_W = NPAD // NW
ACH = 80
ROWS_T = NPAD // NS
BLKS_PW = 80
CPW = BLKS_PW * BLK
NBLK_PAD = NW * BLKS_PW
NCHILD_PAD = NBLK_PAD * BLK

@functools.lru_cache(maxsize=None)
def _mesh():
    return plsc.VectorSubcoreMesh(core_axis_name="c", subcore_axis_name="s",
                                  num_cores=NC, num_subcores=NS)


def _zero_buf(buf):
    z = jnp.zeros((16,), jnp.float32)

    def row(r, _):
        for v in range(H // 16):
            buf[r, pl.ds(v * 16, 16)] = z
        return 0

    lax.fori_loop(0, BLK, row, 0)


def _zero_acc(acc, buf, s):
    _zero_buf(buf)
    for k in range(ROWS_T // BLK):
        pltpu.sync_copy(buf, acc.at[pl.ds(s * ROWS_T + k * BLK, BLK), :])


@functools.lru_cache(maxsize=None)
def _gather_x_kernel():
    return pl.kernel(
        _gather_x_body,
        out_type=jax.ShapeDtypeStruct((NPAD, D), jnp.float32),
        mesh=_mesh(),
        scratch_types=[
            pltpu.VMEM((ROWS_W,), jnp.int32),
            pltpu.VMEM((ROWS_W,), jnp.int32),
            pltpu.VMEM((ACH, D), jnp.float32),
            pltpu.SemaphoreType.DMA,
        ],
    )


def _gather_x_body(emb, sen, sel, x_out, sel_v, senv_v, rows_v, sem):
    c = lax.axis_index("c")
    s = lax.axis_index("s")
    w = s * NC + c
    base = w * ROWS_W
    pltpu.sync_copy(sel.at[pl.ds(base, ROWS_W)], sel_v)
    for k in range(ROWS_W // ACH):
        pltpu.async_copy(
            sen.at[sel_v.at[pl.ds(k * ACH, ACH)]],
            senv_v.at[pl.ds(k * ACH, ACH)],
            sem,
        ).wait()
    for k in range(ROWS_W // ACH):
        pltpu.async_copy(
            emb.at[senv_v.at[pl.ds(k * ACH, ACH)]], rows_v, sem
        ).wait()
        pltpu.sync_copy(rows_v, x_out.at[pl.ds(base + k * ACH, ACH), :])


@functools.lru_cache(maxsize=None)
def _expand_kernel():
    return pl.kernel(
        _expand_body,
        out_type=jax.ShapeDtypeStruct((NCHILD, H), jnp.float32),
        mesh=_mesh(),
        scratch_types=[
            pltpu.VMEM_SHARED((NPAD, H), jnp.float32),
            pltpu.VMEM((CPW,), jnp.int32),
            pltpu.VMEM((BLK, H), jnp.float32),
            pltpu.VMEM((BLK, H), jnp.float32),
            pltpu.SemaphoreType.DMA,
            pltpu.SemaphoreType.DMA,
            pltpu.SemaphoreType.DMA,
            pltpu.SemaphoreType.DMA,
        ],
    )


def _expand_body(fx_all, pid_pad, fxg_out, fx_sh, idx_v, fx0, fx1,
                 sg0, sg1, sw0, sw1):
    c = lax.axis_index("c")
    s = lax.axis_index("s")
    w = s * NC + c
    base = w * BLKS_PW
    nvalid = jnp.clip(NBLK - base, 0, BLKS_PW)
    pltpu.sync_copy(
        fx_all.at[pl.ds(s * ROWS_T, ROWS_T), :],
        fx_sh.at[pl.ds(s * ROWS_T, ROWS_T), :],
    )
    pltpu.sync_copy(pid_pad.at[pl.ds(w * CPW, CPW)], idx_v)
    plsc.subcore_barrier()
    fxb = (fx0, fx1)
    sg = (sg0, sg1)
    sw = (sw0, sw1)
    NBUF = 2
    LEAD = 1

    def gather_desc(t, b):
        return pltpu.make_async_copy(
            fx_sh.at[idx_v.at[pl.ds(t * BLK, BLK)]], fxb[b], sg[b])

    def write_desc(t, b):
        return pltpu.make_async_copy(
            fxb[b], fxg_out.at[pl.ds((base + t) * BLK, BLK), :], sw[b])

    def step(t, b):
        @pl.when(jnp.logical_and(t >= NBUF, t - NBUF < nvalid))
        def _():
            write_desc(t - NBUF, b).wait()

        @pl.when(t < nvalid)
        def _():
            gather_desc(t, b).start()

        bb = (b - LEAD) % NBUF

        @pl.when(jnp.logical_and(t >= LEAD, t - LEAD < nvalid))
        def _():
            gather_desc(t - LEAD, bb).wait()
            write_desc(t - LEAD, bb).start()

    nwave = BLKS_PW + NBUF
    nit = (nwave + NBUF - 1) // NBUF

    def body(j2, _):
        for b in range(NBUF):
            step(j2 * NBUF + b, b)
        return 0

    lax.fori_loop(0, nit, body, 0)


@functools.lru_cache(maxsize=None)
def _segsum_kernel(nblk_local, blks_pw):
    body = functools.partial(_segsum_body, nblk_local, blks_pw)
    return pl.kernel(
        body,
        out_type=jax.ShapeDtypeStruct((NC, NPAD, H), jnp.float32),
        mesh=_mesh(),
        scratch_types=[
            pltpu.VMEM_SHARED((NPAD, H), jnp.float32),
            pltpu.VMEM((blks_pw, 1, BLK), jnp.int32),
            pltpu.VMEM((BLK, H), jnp.float32),
            pltpu.VMEM((BLK, H), jnp.float32),
            pltpu.SemaphoreType.DMA,
            pltpu.SemaphoreType.DMA,
            pltpu.SemaphoreType.DMA,
            pltpu.SemaphoreType.DMA,
        ],
    )


def _segsum_body(nblk_local, blks_pw, data, pid3d, out, acc, idx2d, d0, d1,
                 si0, si1, sa0, sa1):
    c = lax.axis_index("c")
    s = lax.axis_index("s")
    w = s * NC + c
    base = w * blks_pw
    nvalid = jnp.clip(nblk_local - base, 0, blks_pw)
    _zero_acc(acc, d0, s)
    pltpu.sync_copy(pid3d.at[pl.ds(base, blks_pw), :, :], idx2d)
    plsc.subcore_barrier()
    bufs = (d0, d1)
    si = (si0, si1)
    sa = (sa0, sa1)
    NBUF = 2

    def load_desc(t, b):
        return pltpu.make_async_copy(
            data.at[pl.ds((base + t) * BLK, BLK), :], bufs[b], si[b])

    def add_desc(t, b):
        return pltpu.make_async_copy(bufs[b], acc.at[idx2d.at[t, 0]], sa[b])

    def step(t, b):
        @pl.when(jnp.logical_and(t >= NBUF, t - NBUF < nvalid))
        def _():
            add_desc(t - NBUF, b).wait()

        @pl.when(t < nvalid)
        def _():
            load_desc(t, b).start()

        bb = (b - 1) % NBUF

        @pl.when(jnp.logical_and(t >= 1, t - 1 < nvalid))
        def _():
            load_desc(t - 1, bb).wait()
            pltpu.async_copy(bufs[bb], acc.at[idx2d.at[t - 1, 0]], sa[bb],
                             add=True)

    nwave = blks_pw + NBUF
    nit = (nwave + NBUF - 1) // NBUF

    def body(j2, _):
        for b in range(NBUF):
            step(j2 * NBUF + b, b)
        return 0

    lax.fori_loop(0, nit, body, 0)
    plsc.subcore_barrier()
    pltpu.sync_copy(
        acc.at[pl.ds(s * ROWS_T, ROWS_T), :],
        out.at[c, pl.ds(s * ROWS_T, ROWS_T), :],
    )


def _mm_t(a, w):
    return lax.dot_general(a, w, (((1,), (1,)), ((), ())),
                           preferred_element_type=jnp.float32)


def _proj_body(x_ref, w_ref, b_ref, o_ref):
    o_ref[...] = _mm_t(x_ref[...], w_ref[...]) + b_ref[...]


def _fc_body(ch_ref, fxg_ref, cc_ref, w_ref, b_ref, fc_ref):
    z = _mm_t(ch_ref[...], w_ref[...]) + b_ref[...] + fxg_ref[...]
    fc_ref[...] = cc_ref[...] * jax.nn.sigmoid(z)


def _gates_body(x_ref, hsp_ref, f0_ref, f1_ref, f2_ref, f3_ref, wix, bix,
                wih, bih, wox, box, woh, boh, wux, bux, wuh, buh, wo, bo,
                out_ref, c_ref, h_ref):
    xs = x_ref[...]
    hs = hsp_ref[0] + hsp_ref[1]
    fcs = ((f0_ref[0] + f0_ref[1]) + (f1_ref[0] + f1_ref[1])
           + (f2_ref[0] + f2_ref[1]) + (f3_ref[0] + f3_ref[1]))
    i = jax.nn.sigmoid(_mm_t(xs, wix[...]) + bix[...]
                       + _mm_t(hs, wih[...]) + bih[...])
    o = jax.nn.sigmoid(_mm_t(xs, wox[...]) + box[...]
                       + _mm_t(hs, woh[...]) + boh[...])
    u = jnp.tanh(_mm_t(xs, wux[...]) + bux[...]
                 + _mm_t(hs, wuh[...]) + buh[...])
    cc = i * u + fcs
    hh = o * jnp.tanh(cc)
    out_ref[...] = _mm_t(hh, wo[...])
    c_ref[...] = cc
    h_ref[...] = hh


_W_SPEC = pl.BlockSpec((H, H), lambda i: (0, 0))
_B_SPEC = pl.BlockSpec((H,), lambda i: (0,))


def _proj(x, w, b):
    blk = 1280
    return pl.pallas_call(
        _proj_body,
        grid=(NPAD // blk,),
        in_specs=[pl.BlockSpec((blk, D), lambda i: (i, 0)), _W_SPEC, _B_SPEC],
        out_specs=pl.BlockSpec((blk, H), lambda i: (i, 0)),
        out_shape=jax.ShapeDtypeStruct((NPAD, H), jnp.float32),
    )(x, w, b)


NQ = 4
CQ = NCHILD // NQ
FC_BLK = 1000
QBLKS = CQ // FC_BLK
NBLK_Q = CQ // BLK
BLKS_PW_Q = 24


def _fc(ch, fxg, cc, w, b, q):
    spec = pl.BlockSpec((FC_BLK, H), lambda i: (i + q * QBLKS, 0))
    return pl.pallas_call(
        _fc_body,
        grid=(QBLKS,),
        in_specs=[spec, spec, spec, _W_SPEC, _B_SPEC],
        out_specs=pl.BlockSpec((FC_BLK, H), lambda i: (i, 0)),
        out_shape=jax.ShapeDtypeStruct((CQ, H), jnp.float32),
    )(ch, fxg, cc, w, b)


def _gates(x, hsp, fcsp_list, wix, bix, wih, bih, wox, box, woh, boh, wux,
           bux, wuh, buh, wo, bo):
    blk = 1280
    row_spec = pl.BlockSpec((blk, H), lambda i: (i, 0))
    part_spec = pl.BlockSpec((NC, blk, H), lambda i: (0, i, 0))
    w_specs = [_W_SPEC, _B_SPEC] * 6 + [_W_SPEC, _B_SPEC]
    return pl.pallas_call(
        _gates_body,
        grid=(NPAD // blk,),
        in_specs=[row_spec, part_spec] + [part_spec] * 4 + w_specs,
        out_specs=(row_spec, row_spec, row_spec),
        out_shape=(
            jax.ShapeDtypeStruct((NPAD, H), jnp.float32),
            jax.ShapeDtypeStruct((NPAD, H), jnp.float32),
            jax.ShapeDtypeStruct((NPAD, H), jnp.float32),
        ),
    )(x, hsp, *fcsp_list, wix, bix, wih, bih, wox, box, woh, boh, wux, bux,
      wuh, buh, wo, bo)


def kernel(embedding, Wix, bix, Wih, bih, Wfx, bfx, Wfh, bfh,
           Wox, box, Woh, boh, Wux, bux, Wuh, buh, Wout, bout,
           child_h, child_c, sen, select_indices, parent_ids):
    sen32 = sen.astype(jnp.int32)
    sel32 = jnp.pad(select_indices.astype(jnp.int32), (0, NPAD - N))
    pid32 = parent_ids.astype(jnp.int32)
    wo_pad = jnp.pad(Wout, ((0, H - Wout.shape[0]), (0, 0)))

    pid_pad = jnp.pad(pid32, (0, NCHILD_PAD - NCHILD))
    pid3d = pid_pad.reshape(NBLK_PAD, 1, BLK)

    hs_parts = _segsum_kernel(NBLK, BLKS_PW)(child_h, pid3d)
    x = _gather_x_kernel()(embedding, sen32, sel32)
    fx_all = _proj(x, Wfx, bfx)
    fxg = _expand_kernel()(fx_all, pid_pad)
    pid3d_big = jnp.pad(pid3d, ((0, NQ * BLKS_PW_Q * 32 - NBLK_PAD), (0, 0),
                                (0, 0)))
    fcs_list = []
    for q in range(NQ):
        fc_q = _fc(child_h, fxg, child_c, Wfh, bfh, q)
        pid3d_q = lax.dynamic_slice_in_dim(pid3d_big, q * NBLK_Q,
                                           BLKS_PW_Q * 32, axis=0)
        fcs_list.append(_segsum_kernel(NBLK_Q, BLKS_PW_Q)(fc_q, pid3d_q))
    out_full, c, h = _gates(x, hs_parts, fcs_list, Wix, bix, Wih, bih,
                            Wox, box, Woh, boh, Wux, bux, Wuh, buh,
                            wo_pad, bout)
    out = out_full[:N, : bout.shape[0]] + bout
    return out, c[:N], h[:N]


# speedup vs baseline: 1.1203x; 1.1110x over previous
import functools

import jax
import jax.numpy as jnp
from jax import lax
from jax.experimental import pallas as pl
from jax.experimental.pallas import tpu as pltpu
from jax.experimental.pallas import tpu_sc as plsc

N = 10000
NPAD = 10240
NCHILD = 320000
H = 128
D = 128
NC = 2
NS = 16
NW = NC * NS
BLK = 128
NBLK = NCHILD // BLK
ROWS_W = NPAD // NW
ACH = 80
ROWS_T = NPAD // NS
BLKS_PW = 80
CPW = BLKS_PW * BLK
NBLK_PAD = NW * BLKS_PW
NCHILD_PAD = NBLK_PAD * BLK

@functools.lru_cache(maxsize=None)
def _mesh():
    return plsc.VectorSubcoreMesh(core_axis_name="c", subcore_axis_name="s",
                                  num_cores=NC, num_subcores=NS)


def _zero_buf(buf):
    z = jnp.zeros((16,), jnp.float32)

    def row(r, _):
        for v in range(H // 16):
            buf[r, pl.ds(v * 16, 16)] = z
        return 0

    lax.fori_loop(0, BLK, row, 0)


def _zero_acc(acc, buf, s):
    _zero_buf(buf)
    for k in range(ROWS_T // BLK):
        pltpu.sync_copy(buf, acc.at[pl.ds(s * ROWS_T + k * BLK, BLK), :])


@functools.lru_cache(maxsize=None)
def _gather_x_kernel():
    return pl.kernel(
        _gather_x_body,
        out_type=jax.ShapeDtypeStruct((NPAD, D), jnp.float32),
        mesh=_mesh(),
        scratch_types=[
            pltpu.VMEM((ROWS_W,), jnp.int32),
            pltpu.VMEM((ROWS_W,), jnp.int32),
            pltpu.VMEM((ACH, D), jnp.float32),
            pltpu.SemaphoreType.DMA,
        ],
    )


def _gather_x_body(emb, sen, sel, x_out, sel_v, senv_v, rows_v, sem):
    c = lax.axis_index("c")
    s = lax.axis_index("s")
    w = s * NC + c
    base = w * ROWS_W
    pltpu.sync_copy(sel.at[pl.ds(base, ROWS_W)], sel_v)
    for k in range(ROWS_W // ACH):
        pltpu.async_copy(
            sen.at[sel_v.at[pl.ds(k * ACH, ACH)]],
            senv_v.at[pl.ds(k * ACH, ACH)],
            sem,
        ).wait()
    for k in range(ROWS_W // ACH):
        pltpu.async_copy(
            emb.at[senv_v.at[pl.ds(k * ACH, ACH)]], rows_v, sem
        ).wait()
        pltpu.sync_copy(rows_v, x_out.at[pl.ds(base + k * ACH, ACH), :])


@functools.lru_cache(maxsize=None)
def _expand_kernel():
    return pl.kernel(
        _expand_body,
        out_type=jax.ShapeDtypeStruct((NCHILD, H), jnp.float32),
        mesh=_mesh(),
        scratch_types=[
            pltpu.VMEM_SHARED((NPAD, H), jnp.float32),
            pltpu.VMEM((CPW,), jnp.int32),
            pltpu.VMEM((BLK, H), jnp.float32),
            pltpu.VMEM((BLK, H), jnp.float32),
            pltpu.SemaphoreType.DMA,
            pltpu.SemaphoreType.DMA,
            pltpu.SemaphoreType.DMA,
            pltpu.SemaphoreType.DMA,
        ],
    )


def _expand_body(fx_all, pid_pad, fxg_out, fx_sh, idx_v, fx0, fx1,
                 sg0, sg1, sw0, sw1):
    c = lax.axis_index("c")
    s = lax.axis_index("s")
    w = s * NC + c
    base = w * BLKS_PW
    nvalid = jnp.clip(NBLK - base, 0, BLKS_PW)
    pltpu.sync_copy(
        fx_all.at[pl.ds(s * ROWS_T, ROWS_T), :],
        fx_sh.at[pl.ds(s * ROWS_T, ROWS_T), :],
    )
    pltpu.sync_copy(pid_pad.at[pl.ds(w * CPW, CPW)], idx_v)
    plsc.subcore_barrier()
    fxb = (fx0, fx1)
    sg = (sg0, sg1)
    sw = (sw0, sw1)
    NBUF = 2
    LEAD = 1

    def gather_desc(t, b):
        return pltpu.make_async_copy(
            fx_sh.at[idx_v.at[pl.ds(t * BLK, BLK)]], fxb[b], sg[b])

    def write_desc(t, b):
        return pltpu.make_async_copy(
            fxb[b], fxg_out.at[pl.ds((base + t) * BLK, BLK), :], sw[b])

    def step(t, b):
        @pl.when(jnp.logical_and(t >= NBUF, t - NBUF < nvalid))
        def _():
            write_desc(t - NBUF, b).wait()

        @pl.when(t < nvalid)
        def _():
            gather_desc(t, b).start()

        bb = (b - LEAD) % NBUF

        @pl.when(jnp.logical_and(t >= LEAD, t - LEAD < nvalid))
        def _():
            gather_desc(t - LEAD, bb).wait()
            write_desc(t - LEAD, bb).start()

    nwave = BLKS_PW + NBUF
    nit = (nwave + NBUF - 1) // NBUF

    def body(j2, _):
        for b in range(NBUF):
            step(j2 * NBUF + b, b)
        return 0

    lax.fori_loop(0, nit, body, 0)


@functools.lru_cache(maxsize=None)
def _segsum_kernel(nblk_local, blks_pw):
    body = functools.partial(_segsum_body, nblk_local, blks_pw)
    return pl.kernel(
        body,
        out_type=jax.ShapeDtypeStruct((NC, NPAD, H), jnp.float32),
        mesh=_mesh(),
        scratch_types=[
            pltpu.VMEM_SHARED((NPAD, H), jnp.float32),
            pltpu.VMEM((blks_pw, 1, BLK), jnp.int32),
            pltpu.VMEM((BLK, H), jnp.float32),
            pltpu.VMEM((BLK, H), jnp.float32),
            pltpu.SemaphoreType.DMA,
            pltpu.SemaphoreType.DMA,
            pltpu.SemaphoreType.DMA,
            pltpu.SemaphoreType.DMA,
        ],
    )


def _segsum_body(nblk_local, blks_pw, data, pid3d, out, acc, idx2d, d0, d1,
                 si0, si1, sa0, sa1):
    c = lax.axis_index("c")
    s = lax.axis_index("s")
    w = s * NC + c
    base = w * blks_pw
    nvalid = jnp.clip(nblk_local - base, 0, blks_pw)
    _zero_acc(acc, d0, s)
    pltpu.sync_copy(pid3d.at[pl.ds(base, blks_pw), :, :], idx2d)
    plsc.subcore_barrier()
    bufs = (d0, d1)
    si = (si0, si1)
    sa = (sa0, sa1)
    NBUF = 2

    def load_desc(t, b):
        return pltpu.make_async_copy(
            data.at[pl.ds((base + t) * BLK, BLK), :], bufs[b], si[b])

    def add_desc(t, b):
        return pltpu.make_async_copy(bufs[b], acc.at[idx2d.at[t, 0]], sa[b])

    def step(t, b):
        @pl.when(jnp.logical_and(t >= NBUF, t - NBUF < nvalid))
        def _():
            add_desc(t - NBUF, b).wait()

        @pl.when(t < nvalid)
        def _():
            load_desc(t, b).start()

        bb = (b - 1) % NBUF

        @pl.when(jnp.logical_and(t >= 1, t - 1 < nvalid))
        def _():
            load_desc(t - 1, bb).wait()
            pltpu.async_copy(bufs[bb], acc.at[idx2d.at[t - 1, 0]], sa[bb],
                             add=True)

    nwave = blks_pw + NBUF
    nit = (nwave + NBUF - 1) // NBUF

    def body(j2, _):
        for b in range(NBUF):
            step(j2 * NBUF + b, b)
        return 0

    lax.fori_loop(0, nit, body, 0)
    plsc.subcore_barrier()
    pltpu.sync_copy(
        acc.at[pl.ds(s * ROWS_T, ROWS_T), :],
        out.at[c, pl.ds(s * ROWS_T, ROWS_T), :],
    )


def _mm_t(a, w):
    return lax.dot_general(a, w, (((1,), (1,)), ((), ())),
                           preferred_element_type=jnp.float32)


def _proj_body(x_ref, w_ref, b_ref, o_ref):
    o_ref[...] = _mm_t(x_ref[...], w_ref[...]) + b_ref[...]


def _fc_body(ch_ref, fxg_ref, cc_ref, w_ref, b_ref, fc_ref):
    z = _mm_t(ch_ref[...], w_ref[...]) + b_ref[...] + fxg_ref[...]
    fc_ref[...] = cc_ref[...] * jax.nn.sigmoid(z)


def _gates_body(x_ref, hsp_ref, f0_ref, f1_ref, wix, bix,
                wih, bih, wox, box, woh, boh, wux, bux, wuh, buh, wo, bo,
                out_ref, c_ref, h_ref):
    xs = x_ref[...]
    hs = hsp_ref[0] + hsp_ref[1]
    fcs = (f0_ref[0] + f0_ref[1]) + (f1_ref[0] + f1_ref[1])
    i = jax.nn.sigmoid(_mm_t(xs, wix[...]) + bix[...]
                       + _mm_t(hs, wih[...]) + bih[...])
    o = jax.nn.sigmoid(_mm_t(xs, wox[...]) + box[...]
                       + _mm_t(hs, woh[...]) + boh[...])
    u = jnp.tanh(_mm_t(xs, wux[...]) + bux[...]
                 + _mm_t(hs, wuh[...]) + buh[...])
    cc = i * u + fcs
    hh = o * jnp.tanh(cc)
    out_ref[...] = _mm_t(hh, wo[...])
    c_ref[...] = cc
    h_ref[...] = hh


_W_SPEC = pl.BlockSpec((H, H), lambda i: (0, 0))
_B_SPEC = pl.BlockSpec((H,), lambda i: (0,))


def _proj(x, w, b):
    blk = 1280
    return pl.pallas_call(
        _proj_body,
        grid=(NPAD // blk,),
        in_specs=[pl.BlockSpec((blk, D), lambda i: (i, 0)), _W_SPEC, _B_SPEC],
        out_specs=pl.BlockSpec((blk, H), lambda i: (i, 0)),
        out_shape=jax.ShapeDtypeStruct((NPAD, H), jnp.float32),
    )(x, w, b)


NQ = 2
CQ = NCHILD // NQ
FC_BLK = 2000
QBLKS = CQ // FC_BLK
NBLK_Q = CQ // BLK
BLKS_PW_Q = 40


def _fc(ch, fxg, cc, w, b, q):
    spec = pl.BlockSpec((FC_BLK, H), lambda i: (i + q * QBLKS, 0))
    return pl.pallas_call(
        _fc_body,
        grid=(QBLKS,),
        in_specs=[spec, spec, spec, _W_SPEC, _B_SPEC],
        out_specs=pl.BlockSpec((FC_BLK, H), lambda i: (i, 0)),
        out_shape=jax.ShapeDtypeStruct((CQ, H), jnp.float32),
    )(ch, fxg, cc, w, b)


def _gates(x, hsp, fcsp_list, wix, bix, wih, bih, wox, box, woh, boh, wux,
           bux, wuh, buh, wo, bo):
    blk = 1280
    row_spec = pl.BlockSpec((blk, H), lambda i: (i, 0))
    part_spec = pl.BlockSpec((NC, blk, H), lambda i: (0, i, 0))
    w_specs = [_W_SPEC, _B_SPEC] * 6 + [_W_SPEC, _B_SPEC]
    return pl.pallas_call(
        _gates_body,
        grid=(NPAD // blk,),
        in_specs=[row_spec, part_spec] + [part_spec] * NQ + w_specs,
        out_specs=(row_spec, row_spec, row_spec),
        out_shape=(
            jax.ShapeDtypeStruct((NPAD, H), jnp.float32),
            jax.ShapeDtypeStruct((NPAD, H), jnp.float32),
            jax.ShapeDtypeStruct((NPAD, H), jnp.float32),
        ),
    )(x, hsp, *fcsp_list, wix, bix, wih, bih, wox, box, woh, boh, wux, bux,
      wuh, buh, wo, bo)


def kernel(embedding, Wix, bix, Wih, bih, Wfx, bfx, Wfh, bfh,
           Wox, box, Woh, boh, Wux, bux, Wuh, buh, Wout, bout,
           child_h, child_c, sen, select_indices, parent_ids):
    sen32 = sen.astype(jnp.int32)
    sel32 = jnp.pad(select_indices.astype(jnp.int32), (0, NPAD - N))
    pid32 = parent_ids.astype(jnp.int32)
    wo_pad = jnp.pad(Wout, ((0, H - Wout.shape[0]), (0, 0)))

    pid_pad = jnp.pad(pid32, (0, NCHILD_PAD - NCHILD))
    pid3d = pid_pad.reshape(NBLK_PAD, 1, BLK)

    hs_parts = _segsum_kernel(NBLK, BLKS_PW)(child_h, pid3d)
    x = _gather_x_kernel()(embedding, sen32, sel32)
    fx_all = _proj(x, Wfx, bfx)
    fxg = _expand_kernel()(fx_all, pid_pad)
    pid3d_big = jnp.pad(pid3d, ((0, NQ * BLKS_PW_Q * 32 - NBLK_PAD), (0, 0),
                                (0, 0)))
    fcs_list = []
    for q in range(NQ):
        fc_q = _fc(child_h, fxg, child_c, Wfh, bfh, q)
        pid3d_q = lax.dynamic_slice_in_dim(pid3d_big, q * NBLK_Q,
                                           BLKS_PW_Q * 32, axis=0)
        fcs_list.append(_segsum_kernel(NBLK_Q, BLKS_PW_Q)(fc_q, pid3d_q))
    out_full, c, h = _gates(x, hs_parts, fcs_list, Wix, bix, Wih, bih,
                            Wox, box, Woh, boh, Wux, bux, Wuh, buh,
                            wo_pad, bout)
    out = out_full[:N, : bout.shape[0]] + bout
    return out, c[:N], h[:N]


# speedup vs baseline: 1.1542x; 1.0303x over previous
import functools

import jax
import jax.numpy as jnp
from jax import lax
from jax.experimental import pallas as pl
from jax.experimental.pallas import tpu as pltpu
from jax.experimental.pallas import tpu_sc as plsc

N = 10000
NPAD = 10240
NCHILD = 320000
H = 128
D = 128
NC = 2
NS = 16
NW = NC * NS
BLK = 128
NBLK = NCHILD // BLK
ROWS_W = NPAD // NW
ACH = 80
ROWS_T = NPAD // NS
BLKS_PW = 80
CPW = BLKS_PW * BLK
NBLK_PAD = NW * BLKS_PW
NCHILD_PAD = NBLK_PAD * BLK

@functools.lru_cache(maxsize=None)
def _mesh():
    return plsc.VectorSubcoreMesh(core_axis_name="c", subcore_axis_name="s",
                                  num_cores=NC, num_subcores=NS)


def _zero_buf(buf):
    z = jnp.zeros((16,), jnp.float32)

    def row(r, _):
        for v in range(H // 16):
            buf[r, pl.ds(v * 16, 16)] = z
        return 0

    lax.fori_loop(0, BLK, row, 0)


def _zero_acc(acc, buf, s):
    _zero_buf(buf)
    for k in range(ROWS_T // BLK):
        pltpu.sync_copy(buf, acc.at[pl.ds(s * ROWS_T + k * BLK, BLK), :])


@functools.lru_cache(maxsize=None)
def _gather_x_kernel():
    return pl.kernel(
        _gather_x_body,
        out_type=jax.ShapeDtypeStruct((NPAD, D), jnp.float32),
        mesh=_mesh(),
        scratch_types=[
            pltpu.VMEM((ROWS_W,), jnp.int32),
            pltpu.VMEM((ROWS_W,), jnp.int32),
            pltpu.VMEM((ACH, D), jnp.float32),
            pltpu.SemaphoreType.DMA,
        ],
    )


def _gather_x_body(emb, sen, sel, x_out, sel_v, senv_v, rows_v, sem):
    c = lax.axis_index("c")
    s = lax.axis_index("s")
    w = s * NC + c
    base = w * ROWS_W
    pltpu.sync_copy(sel.at[pl.ds(base, ROWS_W)], sel_v)
    for k in range(ROWS_W // ACH):
        pltpu.async_copy(
            sen.at[sel_v.at[pl.ds(k * ACH, ACH)]],
            senv_v.at[pl.ds(k * ACH, ACH)],
            sem,
        ).wait()
    for k in range(ROWS_W // ACH):
        pltpu.async_copy(
            emb.at[senv_v.at[pl.ds(k * ACH, ACH)]], rows_v, sem
        ).wait()
        pltpu.sync_copy(rows_v, x_out.at[pl.ds(base + k * ACH, ACH), :])


@functools.lru_cache(maxsize=None)
def _expand_kernel():
    return pl.kernel(
        _expand_body,
        out_type=jax.ShapeDtypeStruct((NCHILD, H), jnp.float32),
        mesh=_mesh(),
        scratch_types=[
            pltpu.VMEM_SHARED((NPAD, H), jnp.float32),
            pltpu.VMEM((CPW,), jnp.int32),
            pltpu.VMEM((BLK, H), jnp.float32),
            pltpu.VMEM((BLK, H), jnp.float32),
            pltpu.SemaphoreType.DMA,
            pltpu.SemaphoreType.DMA,
            pltpu.SemaphoreType.DMA,
            pltpu.SemaphoreType.DMA,
        ],
    )


def _expand_body(fx_all, pid_pad, fxg_out, fx_sh, idx_v, fx0, fx1,
                 sg0, sg1, sw0, sw1):
    c = lax.axis_index("c")
    s = lax.axis_index("s")
    w = s * NC + c
    base = w * BLKS_PW
    nvalid = jnp.clip(NBLK - base, 0, BLKS_PW)
    pltpu.sync_copy(
        fx_all.at[pl.ds(s * ROWS_T, ROWS_T), :],
        fx_sh.at[pl.ds(s * ROWS_T, ROWS_T), :],
    )
    pltpu.sync_copy(pid_pad.at[pl.ds(w * CPW, CPW)], idx_v)
    plsc.subcore_barrier()
    fxb = (fx0, fx1)
    sg = (sg0, sg1)
    sw = (sw0, sw1)
    NBUF = 2
    LEAD = 1

    def gather_desc(t, b):
        return pltpu.make_async_copy(
            fx_sh.at[idx_v.at[pl.ds(t * BLK, BLK)]], fxb[b], sg[b])

    def write_desc(t, b):
        return pltpu.make_async_copy(
            fxb[b], fxg_out.at[pl.ds((base + t) * BLK, BLK), :], sw[b])

    def step(t, b):
        @pl.when(jnp.logical_and(t >= NBUF, t - NBUF < nvalid))
        def _():
            write_desc(t - NBUF, b).wait()

        @pl.when(t < nvalid)
        def _():
            gather_desc(t, b).start()

        bb = (b - LEAD) % NBUF

        @pl.when(jnp.logical_and(t >= LEAD, t - LEAD < nvalid))
        def _():
            gather_desc(t - LEAD, bb).wait()
            write_desc(t - LEAD, bb).start()

    nwave = BLKS_PW + NBUF
    nit = (nwave + NBUF - 1) // NBUF

    def body(j2, _):
        for b in range(NBUF):
            step(j2 * NBUF + b, b)
        return 0

    lax.fori_loop(0, nit, body, 0)


@functools.lru_cache(maxsize=None)
def _segsum_kernel(nblk_local, blks_pw):
    body = functools.partial(_segsum_body, nblk_local, blks_pw)
    return pl.kernel(
        body,
        out_type=jax.ShapeDtypeStruct((NC, NPAD, H), jnp.float32),
        mesh=_mesh(),
        scratch_types=[
            pltpu.VMEM_SHARED((NPAD, H), jnp.float32),
            pltpu.VMEM((blks_pw, 1, BLK), jnp.int32),
            pltpu.VMEM((BLK, H), jnp.float32),
            pltpu.VMEM((BLK, H), jnp.float32),
            pltpu.SemaphoreType.DMA,
            pltpu.SemaphoreType.DMA,
            pltpu.SemaphoreType.DMA,
            pltpu.SemaphoreType.DMA,
        ],
    )


def _segsum_body(nblk_local, blks_pw, data, pid3d, out, acc, idx2d, d0, d1,
                 si0, si1, sa0, sa1):
    c = lax.axis_index("c")
    s = lax.axis_index("s")
    w = s * NC + c
    base = w * blks_pw
    nvalid = jnp.clip(nblk_local - base, 0, blks_pw)
    _zero_acc(acc, d0, s)
    pltpu.sync_copy(pid3d.at[pl.ds(base, blks_pw), :, :], idx2d)
    plsc.subcore_barrier()
    bufs = (d0, d1)
    si = (si0, si1)
    sa = (sa0, sa1)
    NBUF = 2

    def load_desc(t, b):
        return pltpu.make_async_copy(
            data.at[pl.ds((base + t) * BLK, BLK), :], bufs[b], si[b])

    def add_desc(t, b):
        return pltpu.make_async_copy(bufs[b], acc.at[idx2d.at[t, 0]], sa[b])

    def step(t, b):
        @pl.when(jnp.logical_and(t >= NBUF, t - NBUF < nvalid))
        def _():
            add_desc(t - NBUF, b).wait()

        @pl.when(t < nvalid)
        def _():
            load_desc(t, b).start()

        bb = (b - 1) % NBUF

        @pl.when(jnp.logical_and(t >= 1, t - 1 < nvalid))
        def _():
            load_desc(t - 1, bb).wait()
            pltpu.async_copy(bufs[bb], acc.at[idx2d.at[t - 1, 0]], sa[bb],
                             add=True)

    nwave = blks_pw + NBUF
    nit = (nwave + NBUF - 1) // NBUF

    def body(j2, _):
        for b in range(NBUF):
            step(j2 * NBUF + b, b)
        return 0

    lax.fori_loop(0, nit, body, 0)
    plsc.subcore_barrier()
    pltpu.sync_copy(
        acc.at[pl.ds(s * ROWS_T, ROWS_T), :],
        out.at[c, pl.ds(s * ROWS_T, ROWS_T), :],
    )


def _mm_t(a, w):
    return lax.dot_general(a, w, (((1,), (1,)), ((), ())),
                           preferred_element_type=jnp.float32)


def _proj_body(x_ref, w_ref, b_ref, o_ref):
    o_ref[...] = _mm_t(x_ref[...], w_ref[...]) + b_ref[...]


def _fc_body(ch_ref, fxg_ref, cc_ref, w_ref, b_ref, fc_ref):
    z = _mm_t(ch_ref[...], w_ref[...]) + b_ref[...] + fxg_ref[...]
    fc_ref[...] = cc_ref[...] * jax.nn.sigmoid(z)


def _gates_body(x_ref, hsp_ref, f0_ref, f1_ref, wix, bix,
                wih, bih, wox, box, woh, boh, wux, bux, wuh, buh, wo, bo,
                out_ref, c_ref, h_ref):
    xs = x_ref[...]
    hs = hsp_ref[0] + hsp_ref[1]
    fcs = (f0_ref[0] + f0_ref[1]) + (f1_ref[0] + f1_ref[1])
    i = jax.nn.sigmoid(_mm_t(xs, wix[...]) + bix[...]
                       + _mm_t(hs, wih[...]) + bih[...])
    o = jax.nn.sigmoid(_mm_t(xs, wox[...]) + box[...]
                       + _mm_t(hs, woh[...]) + boh[...])
    u = jnp.tanh(_mm_t(xs, wux[...]) + bux[...]
                 + _mm_t(hs, wuh[...]) + buh[...])
    cc = i * u + fcs
    hh = o * jnp.tanh(cc)
    out_ref[...] = _mm_t(hh, wo[...])
    c_ref[...] = cc
    h_ref[...] = hh


_W_SPEC = pl.BlockSpec((H, H), lambda i: (0, 0))
_B_SPEC = pl.BlockSpec((H,), lambda i: (0,))


def _proj(x, w, b):
    blk = 1280
    return pl.pallas_call(
        _proj_body,
        grid=(NPAD // blk,),
        in_specs=[pl.BlockSpec((blk, D), lambda i: (i, 0)), _W_SPEC, _B_SPEC],
        out_specs=pl.BlockSpec((blk, H), lambda i: (i, 0)),
        out_shape=jax.ShapeDtypeStruct((NPAD, H), jnp.float32),
    )(x, w, b)


NQ = 2
CQ = NCHILD // NQ
FC_BLK = 4000
QBLKS = CQ // FC_BLK
NBLK_Q = CQ // BLK
BLKS_PW_Q = 40


def _fc(ch, fxg, cc, w, b, q):
    spec = pl.BlockSpec((FC_BLK, H), lambda i: (i + q * QBLKS, 0))
    return pl.pallas_call(
        _fc_body,
        grid=(QBLKS,),
        in_specs=[spec, spec, spec, _W_SPEC, _B_SPEC],
        out_specs=pl.BlockSpec((FC_BLK, H), lambda i: (i, 0)),
        out_shape=jax.ShapeDtypeStruct((CQ, H), jnp.float32),
    )(ch, fxg, cc, w, b)


def _gates(x, hsp, fcsp_list, wix, bix, wih, bih, wox, box, woh, boh, wux,
           bux, wuh, buh, wo, bo):
    blk = 1280
    row_spec = pl.BlockSpec((blk, H), lambda i: (i, 0))
    part_spec = pl.BlockSpec((NC, blk, H), lambda i: (0, i, 0))
    w_specs = [_W_SPEC, _B_SPEC] * 6 + [_W_SPEC, _B_SPEC]
    return pl.pallas_call(
        _gates_body,
        grid=(NPAD // blk,),
        in_specs=[row_spec, part_spec] + [part_spec] * NQ + w_specs,
        out_specs=(row_spec, row_spec, row_spec),
        out_shape=(
            jax.ShapeDtypeStruct((NPAD, H), jnp.float32),
            jax.ShapeDtypeStruct((NPAD, H), jnp.float32),
            jax.ShapeDtypeStruct((NPAD, H), jnp.float32),
        ),
    )(x, hsp, *fcsp_list, wix, bix, wih, bih, wox, box, woh, boh, wux, bux,
      wuh, buh, wo, bo)


def kernel(embedding, Wix, bix, Wih, bih, Wfx, bfx, Wfh, bfh,
           Wox, box, Woh, boh, Wux, bux, Wuh, buh, Wout, bout,
           child_h, child_c, sen, select_indices, parent_ids):
    sen32 = sen.astype(jnp.int32)
    sel32 = jnp.pad(select_indices.astype(jnp.int32), (0, NPAD - N))
    pid32 = parent_ids.astype(jnp.int32)
    wo_pad = jnp.pad(Wout, ((0, H - Wout.shape[0]), (0, 0)))

    pid_pad = jnp.pad(pid32, (0, NCHILD_PAD - NCHILD))
    pid3d = pid_pad.reshape(NBLK_PAD, 1, BLK)

    hs_parts = _segsum_kernel(NBLK, BLKS_PW)(child_h, pid3d)
    x = _gather_x_kernel()(embedding, sen32, sel32)
    fx_all = _proj(x, Wfx, bfx)
    fxg = _expand_kernel()(fx_all, pid_pad)
    pid3d_big = jnp.pad(pid3d, ((0, NQ * BLKS_PW_Q * 32 - NBLK_PAD), (0, 0),
                                (0, 0)))
    fcs_list = []
    for q in range(NQ):
        fc_q = _fc(child_h, fxg, child_c, Wfh, bfh, q)
        pid3d_q = lax.dynamic_slice_in_dim(pid3d_big, q * NBLK_Q,
                                           BLKS_PW_Q * 32, axis=0)
        fcs_list.append(_segsum_kernel(NBLK_Q, BLKS_PW_Q)(fc_q, pid3d_q))
    out_full, c, h = _gates(x, hs_parts, fcs_list, Wix, bix, Wih, bih,
                            Wox, box, Woh, boh, Wux, bux, Wuh, buh,
                            wo_pad, bout)
    out = out_full[:N, : bout.shape[0]] + bout
    return out, c[:N], h[:N]
